# Initial kernel scaffold; baseline (speedup 1.0000x reference)
import jax
import jax.numpy as jnp
from jax.experimental import pallas as pl


def kernel(x, edge_index, edge_attr, W_lin, b_lin, w_pool, W_mlp, b_mlp):
    raise NotImplementedError("write your pallas kernel here")


# speedup vs baseline: 5.7368x; 5.7368x over previous
import functools

import jax
import jax.numpy as jnp
import numpy as np
from jax.experimental import pallas as pl

N = 10000
NP = 10240
D_IN = 128
D_HID = 256
N_CLUSTERS = 16
ALPHA = 0.3
K_ROUNDS = 20
POOL_K = 1000
CSUM = (1.0 - ALPHA) / K_ROUNDS

ROW_TILE = 256


def _matmul_kernel(a_ref, x_ref, o_ref):
    o_ref[...] = jnp.dot(a_ref[...], x_ref[...],
                         preferred_element_type=jnp.float32)


def _propagate(a, xk):
    return pl.pallas_call(
        _matmul_kernel,
        grid=(NP // ROW_TILE,),
        in_specs=[
            pl.BlockSpec((ROW_TILE, NP), lambda i: (i, 0)),
            pl.BlockSpec((NP, D_IN), lambda i: (0, 0)),
        ],
        out_specs=pl.BlockSpec((ROW_TILE, D_IN), lambda i: (i, 0)),
        out_shape=jax.ShapeDtypeStruct((NP, D_IN), jnp.float32),
    )(a, xk)


def _float_key(f):
    u = jax.lax.bitcast_convert_type(f, jnp.int32)
    return u ^ ((u >> 31) & jnp.int32(0x7FFFFFFF))


def _head_kernel(x_ref, s_ref, wlt_ref, bl_ref, wp_ref, wmt_ref, bm_ref,
                 o_ref):
    hp = ALPHA * x_ref[...] + CSUM * s_ref[...]
    h = jnp.dot(hp, wlt_ref[...], preferred_element_type=jnp.float32)
    h = h + bl_ref[...]
    h = jnp.where(h > 0, h, jnp.exp(h) - 1.0)

    wp = wp_ref[...]
    inv_nrm = 1.0 / jnp.sqrt(jnp.sum(wp * wp))
    score = jnp.tanh(
        jnp.dot(h, wp, preferred_element_type=jnp.float32) * inv_nrm)

    keys = _float_key(score)
    lo0 = _float_key(jnp.float32(-1.0)).astype(jnp.int32)
    hi0 = _float_key(jnp.float32(1.0)).astype(jnp.int32)

    def body(_, lh):
        lo, hi = lh
        d = hi - lo
        mid = lo + (d >> 1) + (d & 1)
        cnt = jnp.sum((keys >= mid).astype(jnp.int32))
        pred = cnt >= POOL_K
        return (jnp.where(pred, mid, lo), jnp.where(pred, hi, mid - 1))

    v, _ = jax.lax.fori_loop(0, 32, body, (lo0, hi0))

    n_gt = jnp.sum((keys > v).astype(jnp.int32))
    n_eq = jnp.sum((keys == v).astype(jnp.int32))
    w_eq = (POOL_K - n_gt).astype(jnp.float32) / n_eq.astype(jnp.float32)
    sel = jnp.where(keys > v, 1.0, jnp.where(keys == v, w_eq, 0.0))

    coef = sel * score * (1.0 / POOL_K)
    pooled = jnp.sum(h * coef, axis=0, keepdims=True)
    o_ref[...] = (jnp.dot(pooled, wmt_ref[...],
                          preferred_element_type=jnp.float32)
                  + bm_ref[...])


@jax.jit
def kernel(x, edge_index, edge_attr, W_lin, b_lin, w_pool, W_mlp, b_mlp):
    row = edge_index[0]
    col = edge_index[1]
    loop = jnp.arange(N, dtype=edge_index.dtype)
    rows = jnp.concatenate([row, loop])
    cols = jnp.concatenate([col, loop])
    ew = jnp.concatenate([edge_attr, jnp.ones((N,), dtype=edge_attr.dtype)])
    deg = jnp.zeros((N,), dtype=edge_attr.dtype).at[cols].add(ew)
    dis = jnp.where(deg > 0, 1.0 / jnp.sqrt(deg), 0.0)
    norm = dis[rows] * ew * dis[cols]

    a = jnp.zeros((NP, NP), dtype=jnp.float32).at[cols, rows].add(norm)

    xp = jnp.zeros((NP, D_IN), dtype=jnp.float32).at[:N].set(x)
    xk = xp
    s = jnp.zeros_like(xp)
    for _ in range(K_ROUNDS):
        xk = _propagate(a, xk)
        s = s + xk

    out = pl.pallas_call(
        _head_kernel,
        out_shape=jax.ShapeDtypeStruct((1, N_CLUSTERS), jnp.float32),
    )(x, s[:N], W_lin.T, b_lin[None, :], w_pool[:, None], W_mlp.T,
      b_mlp[None, :])
    return out
